# baseline (speedup 1.0000x reference)
import jax
import jax.numpy as jnp
from jax.experimental import pallas as pl


def _add_kernel(x_ref, emb_ref, o_ref):
    o_ref[...] = x_ref[...] + emb_ref[...]


def kernel(x, emb):
    B, T, D = x.shape
    RB = 512
    xf = x.reshape(B * T, D)
    nblk_t = T // RB
    out = pl.pallas_call(
        _add_kernel,
        grid=(B * T // RB,),
        in_specs=[
            pl.BlockSpec((RB, D), lambda i: (i, 0)),
            pl.BlockSpec((RB, D), lambda i: (i % nblk_t, 0)),
        ],
        out_specs=pl.BlockSpec((RB, D), lambda i: (i, 0)),
        out_shape=jax.ShapeDtypeStruct((B * T, D), x.dtype),
    )(xf, emb)
    return out.reshape(B, T, D)


# speedup vs baseline: 1.3263x; 1.3263x over previous
import jax
import jax.numpy as jnp
from jax.experimental import pallas as pl


def _add_kernel(x_ref, emb_ref, o_ref):
    o_ref[...] = x_ref[...] + emb_ref[...]


def kernel(x, emb):
    B, T, D = x.shape
    TB = 512
    return pl.pallas_call(
        _add_kernel,
        grid=(T // TB,),
        in_specs=[
            pl.BlockSpec((B, TB, D), lambda i: (0, i, 0)),
            pl.BlockSpec((TB, D), lambda i: (i, 0)),
        ],
        out_specs=pl.BlockSpec((B, TB, D), lambda i: (0, i, 0)),
        out_shape=jax.ShapeDtypeStruct((B, T, D), x.dtype),
    )(x, emb)
